# initial kernel scaffold (unmeasured)
import jax
import jax.numpy as jnp
from jax import lax
from jax.experimental import pallas as pl
from jax.experimental.pallas import tpu as pltpu

N_DEV = 8
SQ = 2048
SKV = 2048
D_MODEL = 1024
DH = 128
H_LOCAL = 8
WIN = 128
QBLK = 256
KSPAN = 512
SCALE = 0.08838834764831843


def _compute_body(x_ref, wq_ref, k_ref, v_ref, wo_ref, out_ref, q_scr, ctx_scr):
    h = pl.program_id(0)

    q_scr[...] = (
        jnp.dot(x_ref[...], wq_ref[...], preferred_element_type=jnp.float32)
        * SCALE
    )

    def qblock(qb, carry):
        start = jnp.clip(qb * QBLK - WIN, 0, SKV - KSPAN)
        qblk = q_scr[pl.ds(qb * QBLK, QBLK), :]
        kblk = k_ref[0, pl.ds(start, KSPAN), :]
        s = lax.dot_general(
            qblk, kblk, (((1,), (1,)), ((), ())),
            preferred_element_type=jnp.float32,
        )
        qi = qb * QBLK + lax.broadcasted_iota(jnp.int32, (QBLK, KSPAN), 0)
        ki = start + lax.broadcasted_iota(jnp.int32, (QBLK, KSPAN), 1)
        mask = jnp.abs(qi - ki) <= WIN
        s = jnp.where(mask, s, -1e9)
        m = jnp.max(s, axis=1, keepdims=True)
        w = jnp.exp(s - m)
        w = w / jnp.sum(w, axis=1, keepdims=True)
        vblk = v_ref[0, pl.ds(start, KSPAN), :]
        ctx_scr[pl.ds(qb * QBLK, QBLK), :] = jnp.dot(
            w, vblk, preferred_element_type=jnp.float32
        )
        return carry

    lax.fori_loop(0, SQ // QBLK, qblock, 0)

    contrib = jnp.dot(
        ctx_scr[...], wo_ref[...], preferred_element_type=jnp.float32
    )

    @pl.when(h == 0)
    def _():
        out_ref[...] = contrib

    @pl.when(h != 0)
    def _():
        out_ref[...] += contrib


def _compute_partial(x2, Wq, K, V, Wo):
    return pl.pallas_call(
        _compute_body,
        grid=(H_LOCAL,),
        in_specs=[
            pl.BlockSpec((SQ, D_MODEL), lambda h: (0, 0)),
            pl.BlockSpec((D_MODEL, DH), lambda h: (0, h)),
            pl.BlockSpec((1, SKV, DH), lambda h: (h, 0, 0)),
            pl.BlockSpec((1, SKV, DH), lambda h: (h, 0, 0)),
            pl.BlockSpec((DH, D_MODEL), lambda h: (h, 0)),
        ],
        out_specs=pl.BlockSpec((SQ, D_MODEL), lambda h: (0, 0)),
        out_shape=jax.ShapeDtypeStruct((SQ, D_MODEL), jnp.float32),
        scratch_shapes=[
            pltpu.VMEM((SQ, DH), jnp.float32),
            pltpu.VMEM((SQ, DH), jnp.float32),
        ],
        compiler_params=pltpu.CompilerParams(
            dimension_semantics=("arbitrary",),
        ),
    )(x2, Wq, K, V, Wo)


_SIZES = (1024, 512, 256)
_COMM_OFF = (0, 1024, 1536)
_COMM_ROWS = 1792


def _allreduce_body(p_ref, out_ref, comm_ref, rs_send, rs_recv, ag_send, ag_recv):
    pos = lax.axis_index("i")
    q = lax.rem(pos, 4)
    zb = pos // 4
    xb = lax.rem((q + 1) // 2, 2)
    yb = q // 2
    partners = [
        zb * 4 + jnp.bitwise_xor(q, 1),
        zb * 4 + (3 - q),
        lax.rem(pos + 4, 8),
    ]
    bits = [xb, yb, zb]

    out_ref[...] = p_ref[...]

    keep = 0
    for s in range(3):
        size = _SIZES[s]
        b = bits[s]
        send_start = keep + (1 - b) * size
        keep = keep + b * size
        rdma = pltpu.make_async_remote_copy(
            src_ref=out_ref.at[pl.ds(send_start, size), :],
            dst_ref=comm_ref.at[pl.ds(_COMM_OFF[s], size), :],
            send_sem=rs_send.at[s],
            recv_sem=rs_recv.at[s],
            device_id=(partners[s],),
            device_id_type=pl.DeviceIdType.MESH,
        )
        rdma.start()
        rdma.wait_recv()
        out_ref[pl.ds(keep, size), :] += comm_ref[
            pl.ds(_COMM_OFF[s], size), :
        ]
        rdma.wait_send()

    cur = keep
    for s in (2, 1, 0):
        size = _SIZES[s]
        b = bits[s]
        rdma = pltpu.make_async_remote_copy(
            src_ref=out_ref.at[pl.ds(cur, size), :],
            dst_ref=out_ref.at[pl.ds(cur, size), :],
            send_sem=ag_send.at[s],
            recv_sem=ag_recv.at[s],
            device_id=(partners[s],),
            device_id_type=pl.DeviceIdType.MESH,
        )
        rdma.start()
        rdma.wait_recv()
        rdma.wait_send()
        cur = cur - b * size


def _allreduce(partial):
    return pl.pallas_call(
        _allreduce_body,
        in_specs=[pl.BlockSpec(memory_space=pltpu.VMEM)],
        out_specs=pl.BlockSpec(memory_space=pltpu.VMEM),
        out_shape=jax.ShapeDtypeStruct((SQ, D_MODEL), jnp.float32),
        scratch_shapes=[
            pltpu.VMEM((_COMM_ROWS, D_MODEL), jnp.float32),
            pltpu.SemaphoreType.DMA((3,)),
            pltpu.SemaphoreType.DMA((3,)),
            pltpu.SemaphoreType.DMA((3,)),
            pltpu.SemaphoreType.DMA((3,)),
        ],
        compiler_params=pltpu.CompilerParams(
            collective_id=0,
            has_side_effects=True,
        ),
    )(partial)


def kernel(x, Wq, K_ext, V_ext, Wo):
    pos = lax.axis_index("i")
    K = jnp.transpose(
        lax.dynamic_slice_in_dim(K_ext[0], pos * H_LOCAL, H_LOCAL, axis=1),
        (1, 0, 2),
    )
    V = jnp.transpose(
        lax.dynamic_slice_in_dim(V_ext[0], pos * H_LOCAL, H_LOCAL, axis=1),
        (1, 0, 2),
    )
    partial = _compute_partial(x[0], Wq, K, V, Wo)
    out = _allreduce(partial)
    return out[None]


# baseline (device time: 275470 ns/iter reference)
import jax
import jax.numpy as jnp
from jax import lax
from jax.experimental import pallas as pl
from jax.experimental.pallas import tpu as pltpu

N_DEV = 8
SQ = 2048
SKV = 2048
D_MODEL = 1024
DH = 128
H_LOCAL = 8
WIN = 128
QBLK = 256
KSPAN = 512
SCALE = 0.08838834764831843


def _compute_body(x_ref, wq_ref, k_ref, v_ref, wo_ref, out_ref, q_scr, ctx_scr):
    h = pl.program_id(0)

    q_scr[...] = (
        jnp.dot(x_ref[...], wq_ref[...], preferred_element_type=jnp.float32)
        * SCALE
    )

    def qblock(qb, carry):
        start = jnp.clip(qb * QBLK - WIN, 0, SKV - KSPAN)
        qblk = q_scr[pl.ds(qb * QBLK, QBLK), :]
        kblk = k_ref[0, pl.ds(start, KSPAN), :]
        s = lax.dot_general(
            qblk, kblk, (((1,), (1,)), ((), ())),
            preferred_element_type=jnp.float32,
        )
        qi = qb * QBLK + lax.broadcasted_iota(jnp.int32, (QBLK, KSPAN), 0)
        ki = start + lax.broadcasted_iota(jnp.int32, (QBLK, KSPAN), 1)
        mask = jnp.abs(qi - ki) <= WIN
        s = jnp.where(mask, s, -1e9)
        m = jnp.max(s, axis=1, keepdims=True)
        w = jnp.exp(s - m)
        w = w / jnp.sum(w, axis=1, keepdims=True)
        vblk = v_ref[0, pl.ds(start, KSPAN), :]
        ctx_scr[pl.ds(qb * QBLK, QBLK), :] = jnp.dot(
            w, vblk, preferred_element_type=jnp.float32
        )
        return carry

    lax.fori_loop(0, SQ // QBLK, qblock, 0)

    contrib = jnp.dot(
        ctx_scr[...], wo_ref[...], preferred_element_type=jnp.float32
    )

    @pl.when(h == 0)
    def _():
        out_ref[...] = contrib

    @pl.when(h != 0)
    def _():
        out_ref[...] += contrib


def _compute_partial(x2, Wq, K, V, Wo):
    return pl.pallas_call(
        _compute_body,
        grid=(H_LOCAL,),
        in_specs=[
            pl.BlockSpec((SQ, D_MODEL), lambda h: (0, 0)),
            pl.BlockSpec((D_MODEL, DH), lambda h: (0, h)),
            pl.BlockSpec((1, SKV, DH), lambda h: (h, 0, 0)),
            pl.BlockSpec((1, SKV, DH), lambda h: (h, 0, 0)),
            pl.BlockSpec((DH, D_MODEL), lambda h: (h, 0)),
        ],
        out_specs=pl.BlockSpec((SQ, D_MODEL), lambda h: (0, 0)),
        out_shape=jax.ShapeDtypeStruct((SQ, D_MODEL), jnp.float32),
        scratch_shapes=[
            pltpu.VMEM((SQ, DH), jnp.float32),
            pltpu.VMEM((SQ, DH), jnp.float32),
        ],
        compiler_params=pltpu.CompilerParams(
            dimension_semantics=("arbitrary",),
        ),
    )(x2, Wq, K, V, Wo)


_SIZES = (1024, 512, 256)
_COMM_OFF = (0, 1024, 1536)
_COMM_ROWS = 1792


def _allreduce_body(p_ref, out_ref, comm_ref, rs_send, rs_recv, ag_send, ag_recv):
    pos = lax.axis_index("i")
    q = lax.rem(pos, 4)
    zb = pos // 4
    xb = lax.rem((q + 1) // 2, 2)
    yb = q // 2
    partners = [
        zb * 4 + jnp.bitwise_xor(q, 1),
        zb * 4 + (3 - q),
        lax.rem(pos + 4, 8),
    ]
    bits = [xb, yb, zb]

    out_ref[...] = p_ref[...]

    keep = 0
    for s in range(3):
        size = _SIZES[s]
        b = bits[s]
        send_start = keep + (1 - b) * size
        keep = keep + b * size
        rdma = pltpu.make_async_remote_copy(
            src_ref=out_ref.at[pl.ds(send_start, size), :],
            dst_ref=comm_ref.at[pl.ds(_COMM_OFF[s], size), :],
            send_sem=rs_send.at[s],
            recv_sem=rs_recv.at[s],
            device_id=(partners[s],),
            device_id_type=pl.DeviceIdType.MESH,
        )
        rdma.start()
        rdma.wait_recv()
        out_ref[pl.ds(keep, size), :] += comm_ref[
            pl.ds(_COMM_OFF[s], size), :
        ]
        rdma.wait_send()

    cur = keep
    for s in (2, 1, 0):
        size = _SIZES[s]
        b = bits[s]
        rdma = pltpu.make_async_remote_copy(
            src_ref=out_ref.at[pl.ds(cur, size), :],
            dst_ref=out_ref.at[pl.ds(cur, size), :],
            send_sem=ag_send.at[s],
            recv_sem=ag_recv.at[s],
            device_id=(partners[s],),
            device_id_type=pl.DeviceIdType.MESH,
        )
        rdma.start()
        rdma.wait_recv()
        rdma.wait_send()
        cur = cur - b * size


def _allreduce(partial):
    return pl.pallas_call(
        _allreduce_body,
        in_specs=[pl.BlockSpec(memory_space=pltpu.VMEM)],
        out_specs=pl.BlockSpec(memory_space=pltpu.VMEM),
        out_shape=jax.ShapeDtypeStruct((SQ, D_MODEL), jnp.float32),
        scratch_shapes=[
            pltpu.VMEM((_COMM_ROWS, D_MODEL), jnp.float32),
            pltpu.SemaphoreType.DMA((3,)),
            pltpu.SemaphoreType.DMA((3,)),
            pltpu.SemaphoreType.DMA((3,)),
            pltpu.SemaphoreType.DMA((3,)),
        ],
        compiler_params=pltpu.CompilerParams(
            has_side_effects=True,
        ),
    )(partial)


def kernel(x, Wq, K_ext, V_ext, Wo):
    pos = lax.axis_index("i")
    K = jnp.transpose(
        lax.dynamic_slice_in_dim(K_ext[0], pos * H_LOCAL, H_LOCAL, axis=1),
        (1, 0, 2),
    )
    V = jnp.transpose(
        lax.dynamic_slice_in_dim(V_ext[0], pos * H_LOCAL, H_LOCAL, axis=1),
        (1, 0, 2),
    )
    partial = _compute_partial(x[0], Wq, K, V, Wo)
    out = _allreduce(partial)
    return out[None]


# device time: 197124 ns/iter; 1.3974x vs baseline; 1.3974x over previous
import jax
import jax.numpy as jnp
from jax import lax
from jax.experimental import pallas as pl
from jax.experimental.pallas import tpu as pltpu

N_DEV = 8
SQ = 2048
SKV = 2048
D_MODEL = 1024
DH = 128
H_LOCAL = 8
WIN = 128
QBLK = 256
KSPAN = 512
SCALE = 0.08838834764831843


def _compute_body(x_ref, wq_ref, k_ref, v_ref, wo_ref, out_ref, q_scr, ctx_scr):
    h = pl.program_id(0)

    q_scr[...] = (
        jnp.dot(x_ref[...], wq_ref[...], preferred_element_type=jnp.float32)
        * SCALE
    )

    def qblock(qb, carry):
        start = jnp.clip(qb * 2 - 1, 0, (SKV - KSPAN) // 128) * 128
        qblk = q_scr[pl.ds(qb * QBLK, QBLK), :].astype(jnp.bfloat16)
        kblk = k_ref[0, pl.ds(start, KSPAN), :]
        s = lax.dot_general(
            qblk, kblk, (((1,), (1,)), ((), ())),
            preferred_element_type=jnp.float32,
        )
        qi = qb * QBLK + lax.broadcasted_iota(jnp.int32, (QBLK, KSPAN), 0)
        ki = start + lax.broadcasted_iota(jnp.int32, (QBLK, KSPAN), 1)
        mask = jnp.abs(qi - ki) <= WIN
        s = jnp.where(mask, s, -1e9)
        m = jnp.max(s, axis=1, keepdims=True)
        w = jnp.exp(s - m)
        w = (w / jnp.sum(w, axis=1, keepdims=True)).astype(jnp.bfloat16)
        vblk = v_ref[0, pl.ds(start, KSPAN), :]
        ctx_scr[pl.ds(qb * QBLK, QBLK), :] = jnp.dot(
            w, vblk, preferred_element_type=jnp.float32
        )
        return carry

    lax.fori_loop(0, SQ // QBLK, qblock, 0)

    contrib = jnp.dot(
        ctx_scr[...].astype(jnp.bfloat16),
        wo_ref[...],
        preferred_element_type=jnp.float32,
    )

    @pl.when(h == 0)
    def _():
        out_ref[...] = contrib

    @pl.when(h != 0)
    def _():
        out_ref[...] += contrib


def _compute_partial(x2, Wq, K, V, Wo):
    return pl.pallas_call(
        _compute_body,
        grid=(H_LOCAL,),
        in_specs=[
            pl.BlockSpec((SQ, D_MODEL), lambda h: (0, 0)),
            pl.BlockSpec((D_MODEL, DH), lambda h: (0, h)),
            pl.BlockSpec((1, SKV, DH), lambda h: (h, 0, 0)),
            pl.BlockSpec((1, SKV, DH), lambda h: (h, 0, 0)),
            pl.BlockSpec((DH, D_MODEL), lambda h: (h, 0)),
        ],
        out_specs=pl.BlockSpec((SQ, D_MODEL), lambda h: (0, 0)),
        out_shape=jax.ShapeDtypeStruct((SQ, D_MODEL), jnp.float32),
        scratch_shapes=[
            pltpu.VMEM((SQ, DH), jnp.float32),
            pltpu.VMEM((SQ, DH), jnp.float32),
        ],
        compiler_params=pltpu.CompilerParams(
            dimension_semantics=("arbitrary",),
        ),
    )(x2, Wq, K, V, Wo)


_HALF_BASE = (0, 1024)
_HALF_ORDER = ((0, 1, 2), (1, 2, 0))
_STAGE_ROWS = (512, 256, 128)
_COMM_OFF = (0, 512, 768)
_COMM_HALF = 896


def _allreduce_body(p_ref, out_ref, comm_ref, rs_send, rs_recv, ag_send, ag_recv):
    pos = lax.axis_index("i")
    q = lax.rem(pos, 4)
    zb = pos // 4
    xb = lax.rem((q + 1) // 2, 2)
    yb = q // 2
    partners = [
        zb * 4 + jnp.bitwise_xor(q, 1),
        zb * 4 + (3 - q),
        lax.rem(pos + 4, 8),
    ]
    bits = [xb, yb, zb]

    out_ref[...] = p_ref[...]

    keep = [jnp.int32(_HALF_BASE[0]), jnp.int32(_HALF_BASE[1])]
    for s in range(3):
        size = _STAGE_ROWS[s]
        rdmas = []
        for hi in range(2):
            d = _HALF_ORDER[hi][s]
            b = bits[d]
            send_start = keep[hi] + (1 - b) * size
            keep[hi] = keep[hi] + b * size
            rdma = pltpu.make_async_remote_copy(
                src_ref=out_ref.at[pl.ds(send_start, size), :],
                dst_ref=comm_ref.at[
                    pl.ds(hi * _COMM_HALF + _COMM_OFF[s], size), :
                ],
                send_sem=rs_send.at[hi, s],
                recv_sem=rs_recv.at[hi, s],
                device_id=(partners[d],),
                device_id_type=pl.DeviceIdType.MESH,
            )
            rdma.start()
            rdmas.append(rdma)
        for hi in range(2):
            rdmas[hi].wait_recv()
            out_ref[pl.ds(keep[hi], size), :] += comm_ref[
                pl.ds(hi * _COMM_HALF + _COMM_OFF[s], size), :
            ]
            rdmas[hi].wait_send()

    cur = keep
    for s in (2, 1, 0):
        size = _STAGE_ROWS[s]
        rdmas = []
        for hi in range(2):
            rdma = pltpu.make_async_remote_copy(
                src_ref=out_ref.at[pl.ds(cur[hi], size), :],
                dst_ref=out_ref.at[pl.ds(cur[hi], size), :],
                send_sem=ag_send.at[hi, s],
                recv_sem=ag_recv.at[hi, s],
                device_id=(partners[_HALF_ORDER[hi][s]],),
                device_id_type=pl.DeviceIdType.MESH,
            )
            rdma.start()
            rdmas.append(rdma)
        for hi in range(2):
            rdmas[hi].wait_recv()
            rdmas[hi].wait_send()
            cur[hi] = cur[hi] - bits[_HALF_ORDER[hi][s]] * size


def _allreduce(partial):
    return pl.pallas_call(
        _allreduce_body,
        in_specs=[pl.BlockSpec(memory_space=pltpu.VMEM)],
        out_specs=pl.BlockSpec(memory_space=pltpu.VMEM),
        out_shape=jax.ShapeDtypeStruct((SQ, D_MODEL), jnp.float32),
        scratch_shapes=[
            pltpu.VMEM((2 * _COMM_HALF, D_MODEL), jnp.float32),
            pltpu.SemaphoreType.DMA((2, 3)),
            pltpu.SemaphoreType.DMA((2, 3)),
            pltpu.SemaphoreType.DMA((2, 3)),
            pltpu.SemaphoreType.DMA((2, 3)),
        ],
        compiler_params=pltpu.CompilerParams(
            has_side_effects=True,
        ),
    )(partial)


def kernel(x, Wq, K_ext, V_ext, Wo):
    pos = lax.axis_index("i")
    K = jnp.transpose(
        lax.dynamic_slice_in_dim(K_ext[0], pos * H_LOCAL, H_LOCAL, axis=1),
        (1, 0, 2),
    ).astype(jnp.bfloat16)
    V = jnp.transpose(
        lax.dynamic_slice_in_dim(V_ext[0], pos * H_LOCAL, H_LOCAL, axis=1),
        (1, 0, 2),
    ).astype(jnp.bfloat16)
    partial = _compute_partial(
        x[0].astype(jnp.bfloat16),
        Wq.astype(jnp.bfloat16),
        K,
        V,
        Wo.astype(jnp.bfloat16),
    )
    out = _allreduce(partial)
    return out[None]


# device time: 176096 ns/iter; 1.5643x vs baseline; 1.1194x over previous
import jax
import jax.numpy as jnp
from jax import lax
from jax.experimental import pallas as pl
from jax.experimental.pallas import tpu as pltpu

N_DEV = 8
SQ = 2048
SKV = 2048
D_MODEL = 1024
DH = 128
H_LOCAL = 8
WIN = 128
QBLK = 256
KSPAN = 512
SCALE = 0.08838834764831843


def _compute_body(x_ref, wq_ref, k_ref, v_ref, wo_ref, out_ref, q_scr, ctx_scr):
    h = pl.program_id(0)

    q_scr[...] = (
        jnp.dot(x_ref[...], wq_ref[...], preferred_element_type=jnp.float32)
        * SCALE
    )

    def qblock(qb, carry):
        start = jnp.clip(qb * 2 - 1, 0, (SKV - KSPAN) // 128) * 128
        qblk = q_scr[pl.ds(qb * QBLK, QBLK), :].astype(jnp.bfloat16)
        kblk = k_ref[0, pl.ds(start, KSPAN), :]
        s = lax.dot_general(
            qblk, kblk, (((1,), (1,)), ((), ())),
            preferred_element_type=jnp.float32,
        )
        qi = qb * QBLK + lax.broadcasted_iota(jnp.int32, (QBLK, KSPAN), 0)
        ki = start + lax.broadcasted_iota(jnp.int32, (QBLK, KSPAN), 1)
        mask = jnp.abs(qi - ki) <= WIN
        s = jnp.where(mask, s, -1e9)
        m = jnp.max(s, axis=1, keepdims=True)
        w = jnp.exp(s - m)
        w = (w / jnp.sum(w, axis=1, keepdims=True)).astype(jnp.bfloat16)
        vblk = v_ref[0, pl.ds(start, KSPAN), :]
        ctx_scr[pl.ds(qb * QBLK, QBLK), :] = jnp.dot(
            w, vblk, preferred_element_type=jnp.float32
        )
        return carry

    lax.fori_loop(0, SQ // QBLK, qblock, 0)

    contrib = jnp.dot(
        ctx_scr[...].astype(jnp.bfloat16),
        wo_ref[...],
        preferred_element_type=jnp.float32,
    )

    @pl.when(h == 0)
    def _():
        out_ref[...] = contrib

    @pl.when(h != 0)
    def _():
        out_ref[...] += contrib


def _compute_partial(x2, Wq, K, V, Wo):
    return pl.pallas_call(
        _compute_body,
        grid=(H_LOCAL,),
        in_specs=[
            pl.BlockSpec((SQ, D_MODEL), lambda h: (0, 0)),
            pl.BlockSpec((D_MODEL, DH), lambda h: (0, h)),
            pl.BlockSpec((1, SKV, DH), lambda h: (h, 0, 0)),
            pl.BlockSpec((1, SKV, DH), lambda h: (h, 0, 0)),
            pl.BlockSpec((DH, D_MODEL), lambda h: (h, 0)),
        ],
        out_specs=pl.BlockSpec((SQ, D_MODEL), lambda h: (0, 0)),
        out_shape=jax.ShapeDtypeStruct((SQ, D_MODEL), jnp.float32),
        scratch_shapes=[
            pltpu.VMEM((SQ, DH), jnp.float32),
            pltpu.VMEM((SQ, DH), jnp.float32),
        ],
        compiler_params=pltpu.CompilerParams(
            dimension_semantics=("arbitrary",),
        ),
    )(x2, Wq, K, V, Wo)


_GROUPS = (
    (0, 768, (0, 1, 2)),
    (768, 640, (1, 2, 0)),
    (1408, 640, (2, 0, 1)),
)
_NG = len(_GROUPS)
_COMM_OFF = []
_off = 0
for _base, _rows, _order in _GROUPS:
    offs = []
    for _s in range(3):
        offs.append(_off)
        _off += _rows >> (_s + 1)
    _COMM_OFF.append(tuple(offs))
_COMM_ROWS = _off


def _allreduce_body(p_ref, out_ref, comm_ref, rs_send, rs_recv, ag_send, ag_recv):
    pos = lax.axis_index("i")
    q = lax.rem(pos, 4)
    zb = pos // 4
    xb = lax.rem((q + 1) // 2, 2)
    yb = q // 2
    partners = [
        zb * 4 + jnp.bitwise_xor(q, 1),
        zb * 4 + (3 - q),
        lax.rem(pos + 4, 8),
    ]
    bits = [xb, yb, zb]

    out_ref[...] = p_ref[...]

    keep = [jnp.int32(g[0]) for g in _GROUPS]
    for s in range(3):
        rdmas = []
        for gi, (base, rows, order) in enumerate(_GROUPS):
            size = rows >> (s + 1)
            d = order[s]
            b = bits[d]
            send_start = keep[gi] + (1 - b) * size
            keep[gi] = keep[gi] + b * size
            rdma = pltpu.make_async_remote_copy(
                src_ref=out_ref.at[pl.ds(send_start, size), :],
                dst_ref=comm_ref.at[pl.ds(_COMM_OFF[gi][s], size), :],
                send_sem=rs_send.at[gi, s],
                recv_sem=rs_recv.at[gi, s],
                device_id=(partners[d],),
                device_id_type=pl.DeviceIdType.MESH,
            )
            rdma.start()
            rdmas.append(rdma)
        for gi, (base, rows, order) in enumerate(_GROUPS):
            size = rows >> (s + 1)
            rdmas[gi].wait_recv()
            out_ref[pl.ds(keep[gi], size), :] += comm_ref[
                pl.ds(_COMM_OFF[gi][s], size), :
            ]
            rdmas[gi].wait_send()

    cur = keep
    for s in (2, 1, 0):
        rdmas = []
        for gi, (base, rows, order) in enumerate(_GROUPS):
            size = rows >> (s + 1)
            rdma = pltpu.make_async_remote_copy(
                src_ref=out_ref.at[pl.ds(cur[gi], size), :],
                dst_ref=out_ref.at[pl.ds(cur[gi], size), :],
                send_sem=ag_send.at[gi, s],
                recv_sem=ag_recv.at[gi, s],
                device_id=(partners[order[s]],),
                device_id_type=pl.DeviceIdType.MESH,
            )
            rdma.start()
            rdmas.append(rdma)
        for gi, (base, rows, order) in enumerate(_GROUPS):
            size = rows >> (s + 1)
            rdmas[gi].wait_recv()
            rdmas[gi].wait_send()
            cur[gi] = cur[gi] - bits[order[s]] * size


def _allreduce(partial):
    return pl.pallas_call(
        _allreduce_body,
        in_specs=[pl.BlockSpec(memory_space=pltpu.VMEM)],
        out_specs=pl.BlockSpec(memory_space=pltpu.VMEM),
        out_shape=jax.ShapeDtypeStruct((SQ, D_MODEL), jnp.float32),
        scratch_shapes=[
            pltpu.VMEM((_COMM_ROWS, D_MODEL), jnp.float32),
            pltpu.SemaphoreType.DMA((_NG, 3)),
            pltpu.SemaphoreType.DMA((_NG, 3)),
            pltpu.SemaphoreType.DMA((_NG, 3)),
            pltpu.SemaphoreType.DMA((_NG, 3)),
        ],
        compiler_params=pltpu.CompilerParams(
            has_side_effects=True,
        ),
    )(partial)


def kernel(x, Wq, K_ext, V_ext, Wo):
    pos = lax.axis_index("i")
    K = jnp.transpose(
        lax.dynamic_slice_in_dim(K_ext[0], pos * H_LOCAL, H_LOCAL, axis=1),
        (1, 0, 2),
    ).astype(jnp.bfloat16)
    V = jnp.transpose(
        lax.dynamic_slice_in_dim(V_ext[0], pos * H_LOCAL, H_LOCAL, axis=1),
        (1, 0, 2),
    ).astype(jnp.bfloat16)
    partial = _compute_partial(
        x[0].astype(jnp.bfloat16),
        Wq.astype(jnp.bfloat16),
        K,
        V,
        Wo.astype(jnp.bfloat16),
    )
    out = _allreduce(partial)
    return out[None]
